# Initial kernel scaffold; baseline (speedup 1.0000x reference)
import jax
import jax.numpy as jnp
from jax.experimental import pallas as pl


def kernel(x, edge_index, input_idx, W_embed, b_embed, bn_gamma, bn_beta, bn_mean, bn_var, arma_Winit, arma_Wroot, arma_bias, head_W1, head_b1, head_W2, head_b2, head_W3, head_b3):
    raise NotImplementedError("write your pallas kernel here")


# speedup vs baseline: 5.1481x; 5.1481x over previous
import functools

import jax
import jax.numpy as jnp
from jax import lax
from jax.experimental import pallas as pl
from jax.experimental.pallas import tpu as pltpu
from jax.experimental.pallas import tpu_sc as plsc

N = 10000
E = 320000
D_IN = 128
D = 256
H = 128
MIX = 10
LAYER_N = 4

NC, NS = 2, 16
CH = 128
AGG_CHUNKS = 158
E_PAD = NS * AGG_CHUNKS * CH
DEG_CHUNKS = E_PAD // (NC * NS * CH)
N_PAD = 10112
ROWS_TILE = N_PAD // NS

BLK = 400
GRID = N // BLK

_mesh = plsc.VectorSubcoreMesh(core_axis_name="c", subcore_axis_name="s")


@functools.partial(
    pl.kernel,
    out_type=jax.ShapeDtypeStruct((NC, N_PAD, H), jnp.float32),
    mesh=_mesh,
    scratch_types=[
        pltpu.VMEM((CH,), jnp.int32),
        pltpu.VMEM((CH,), jnp.int32),
        pltpu.VMEM((CH, H), jnp.float32),
        pltpu.VMEM_SHARED((N_PAD, H), jnp.float32),
        pltpu.SemaphoreType.DMA,
    ],
)
def _sc_agg(rows_hbm, cols_hbm, mp_hbm, zeros_hbm, out_hbm,
            ridx, cidx, gbuf, acc, sem):
    c = lax.axis_index("c")
    s = lax.axis_index("s")
    w = c * NS + s
    pltpu.sync_copy(zeros_hbm.at[pl.ds(s * ROWS_TILE, ROWS_TILE)],
                    acc.at[pl.ds(s * ROWS_TILE, ROWS_TILE)])
    plsc.subcore_barrier()
    rbase = w * (AGG_CHUNKS * CH)
    cbase = s * (AGG_CHUNKS * CH)

    def chunk(j, carry):
        off = pl.multiple_of(j * CH, CH)
        pltpu.sync_copy(rows_hbm.at[pl.ds(rbase + off, CH)], ridx)
        pltpu.sync_copy(cols_hbm.at[pl.ds(cbase + off, CH)], cidx)
        pltpu.async_copy(mp_hbm.at[ridx], gbuf, sem).wait()
        pltpu.sync_copy(gbuf, acc.at[cidx], add=True)
        return carry

    lax.fori_loop(0, AGG_CHUNKS, chunk, 0)
    plsc.subcore_barrier()
    pltpu.sync_copy(acc.at[pl.ds(s * ROWS_TILE, ROWS_TILE)],
                    out_hbm.at[c, pl.ds(s * ROWS_TILE, ROWS_TILE)])


def _lin_body(x_ref, w_ref, b_ref, o_ref):
    o_ref[...] = x_ref[...] @ w_ref[...] + b_ref[...]


def _linear(x, w, b):
    k = x.shape[1]
    return pl.pallas_call(
        _lin_body,
        grid=(GRID,),
        in_specs=[
            pl.BlockSpec((BLK, k), lambda i: (i, 0)),
            pl.BlockSpec((k, D), lambda i: (0, 0)),
            pl.BlockSpec((1, D), lambda i: (0, 0)),
        ],
        out_specs=pl.BlockSpec((BLK, D), lambda i: (i, 0)),
        out_shape=jax.ShapeDtypeStruct((N, D), jnp.float32),
    )(x, w, b.reshape(1, D))


def _mp_body(h_ref, dinv_ref, w_ref, o_ref):
    m = (h_ref[...] @ w_ref[...]) * dinv_ref[...]
    o_ref[0] = m[:, :H]
    o_ref[1] = m[:, H:]


def _mp(h, dinv2, w):
    return pl.pallas_call(
        _mp_body,
        grid=(GRID,),
        in_specs=[
            pl.BlockSpec((BLK, D), lambda i: (i, 0)),
            pl.BlockSpec((BLK, 1), lambda i: (i, 0)),
            pl.BlockSpec((D, D), lambda i: (0, 0)),
        ],
        out_specs=pl.BlockSpec((NC, BLK, H), lambda i: (0, i, 0)),
        out_shape=jax.ShapeDtypeStruct((NC, N, H), jnp.float32),
    )(h, dinv2, w)


def _hnew_body(s_ref, r_ref, dinv_ref, o_ref):
    s_cat = jnp.concatenate([s_ref[0], s_ref[1]], axis=1)
    o_ref[...] = jnp.maximum(s_cat * dinv_ref[...] + r_ref[...], 0.0)


def _hnew(s, r, dinv2):
    return pl.pallas_call(
        _hnew_body,
        grid=(GRID,),
        in_specs=[
            pl.BlockSpec((NC, BLK, H), lambda i: (0, i, 0)),
            pl.BlockSpec((BLK, D), lambda i: (i, 0)),
            pl.BlockSpec((BLK, 1), lambda i: (i, 0)),
        ],
        out_specs=pl.BlockSpec((BLK, D), lambda i: (i, 0)),
        out_shape=jax.ShapeDtypeStruct((N, D), jnp.float32),
    )(s, r, dinv2)


def _heads_body(h_ref, w1_ref, b1_ref, w2_ref, b2_ref, w3_ref, b3_ref,
                mu_ref, std_ref):
    h = h_ref[...]
    ys = []
    for m in range(MIX):
        y1 = jnp.maximum(h @ w1_ref[m] + b1_ref[m], 0.0)
        y2 = jnp.maximum(y1 @ w2_ref[m] + b2_ref[m], 0.0)
        y3 = jnp.sum(y2 * w3_ref[m][None, :], axis=1, keepdims=True)
        ys.append(y3 + b3_ref[m, 0:1])
    mu = ys[0]
    for m in range(1, MIX):
        mu = mu + ys[m]
    mu = mu * (1.0 / MIX)
    var = (ys[0] - mu) ** 2
    for m in range(1, MIX):
        var = var + (ys[m] - mu) ** 2
    std = jnp.sqrt(var * (1.0 / MIX)) + 1e-5
    mu_ref[...] = mu
    std_ref[...] = std


def _heads(h, w1, b1, w2, b2, w3, b3):
    return pl.pallas_call(
        _heads_body,
        grid=(GRID,),
        in_specs=[
            pl.BlockSpec((BLK, D), lambda i: (i, 0)),
            pl.BlockSpec((MIX, D, D), lambda i: (0, 0, 0)),
            pl.BlockSpec((MIX, D), lambda i: (0, 0)),
            pl.BlockSpec((MIX, D, D), lambda i: (0, 0, 0)),
            pl.BlockSpec((MIX, D), lambda i: (0, 0)),
            pl.BlockSpec((MIX, D), lambda i: (0, 0)),
            pl.BlockSpec((MIX, 1), lambda i: (0, 0)),
        ],
        out_specs=[
            pl.BlockSpec((BLK, 1), lambda i: (i, 0)),
            pl.BlockSpec((BLK, 1), lambda i: (i, 0)),
        ],
        out_shape=[
            jax.ShapeDtypeStruct((N, 1), jnp.float32),
            jax.ShapeDtypeStruct((N, 1), jnp.float32),
        ],
    )(h, w1, b1, w2, b2, w3, b3)


def kernel(x, edge_index, input_idx, W_embed, b_embed, bn_gamma, bn_beta,
           bn_mean, bn_var, arma_Winit, arma_Wroot, arma_bias,
           head_W1, head_b1, head_W2, head_b2, head_W3, head_b3):
    f32 = jnp.float32
    row = edge_index[0]
    col = edge_index[1]

    scale = bn_gamma / jnp.sqrt(bn_var + 1e-5)
    w_emb = W_embed * scale[None, :]
    b_emb = (b_embed - bn_mean) * scale + bn_beta

    pad = E_PAD - E
    ar = jnp.arange(pad, dtype=jnp.int32)
    rows_p = jnp.concatenate([row, (ar * 97) % N])
    cols_p = jnp.concatenate([col, N + (ar % 16)])
    rows2 = jnp.stack([rows_p, rows_p + N]).reshape(NC * E_PAD)
    zeros_h = jnp.zeros((N_PAD, H), f32)

    h = _linear(x, w_emb, b_emb)

    deg_parts = _sc_agg(rows2, cols_p, jnp.ones((NC * N, H), f32), zeros_h)
    deg = deg_parts[0, :N, 0]
    dinv2 = jnp.where(deg > 0, deg ** -0.5, 0.0).reshape(N, 1)

    for li in range(LAYER_N):
        mp = _mp(h, dinv2, arma_Winit[li])
        s = _sc_agg(rows2, cols_p, mp.reshape(NC * N, H), zeros_h)
        r = _linear(h, arma_Wroot[li], arma_bias[li])
        h = _hnew(s[:, :N, :], r, dinv2)

    mu, std = _heads(h, head_W1, head_b1, head_W2, head_b2,
                     head_W3.reshape(MIX, D), head_b3.reshape(MIX, 1))
    return mu.reshape(N // 50, 50, 1), std.reshape(N // 50, 50, 1)


# speedup vs baseline: 9.8002x; 1.9036x over previous
import functools

import jax
import jax.numpy as jnp
from jax import lax
from jax.experimental import pallas as pl
from jax.experimental.pallas import tpu as pltpu
from jax.experimental.pallas import tpu_sc as plsc

N = 10000
E = 320000
D_IN = 128
D = 256
H = 128
MIX = 10
LAYER_N = 4

NC, NS = 2, 16
CH = 112
AGG_CHUNKS = 184
E_TILE = AGG_CHUNKS * CH
E_PAD = NS * E_TILE
DEG_CHUNKS = E_PAD // (NC * NS * CH)
N_PAD = 10112
ROWS_TILE = N_PAD // NS

BLK = 400
GRID = N // BLK

_mesh = plsc.VectorSubcoreMesh(core_axis_name="c", subcore_axis_name="s")


GB = 3
IB = 4


@functools.partial(
    pl.kernel,
    out_type=jax.ShapeDtypeStruct((NC, N_PAD, H), jnp.float32),
    mesh=_mesh,
    scratch_types=(
        [pltpu.VMEM((CH,), jnp.int32)] * IB
        + [pltpu.VMEM((CH,), jnp.int32)] * IB
        + [pltpu.VMEM((CH, H), jnp.float32)] * GB
        + [pltpu.VMEM_SHARED((N_PAD, H), jnp.float32)]
        + [pltpu.SemaphoreType.DMA] * (2 * IB + 2 * GB)
    ),
)
def _sc_agg(rows_hbm, cols_hbm, mp_hbm, zeros_hbm, out_hbm, *scr):
    rb = scr[0:IB]
    cb = scr[IB:2 * IB]
    gb = scr[2 * IB:2 * IB + GB]
    acc = scr[2 * IB + GB]
    sems = scr[2 * IB + GB + 1:]
    sir = sems[0:IB]
    sic = sems[IB:2 * IB]
    sg = sems[2 * IB:2 * IB + GB]
    ss = sems[2 * IB + GB:]
    c = lax.axis_index("c")
    s = lax.axis_index("s")
    w = c * NS + s
    rbase = w * E_TILE
    cbase = s * E_TILE
    pltpu.sync_copy(zeros_hbm.at[pl.ds(s * ROWS_TILE, ROWS_TILE)],
                    acc.at[pl.ds(s * ROWS_TILE, ROWS_TILE)])
    plsc.subcore_barrier()

    def i_start(j, bi):
        off = pl.multiple_of(j * CH, CH)
        pltpu.async_copy(rows_hbm.at[pl.ds(rbase + off, CH)], rb[bi], sir[bi])
        pltpu.async_copy(cols_hbm.at[pl.ds(cbase + off, CH)], cb[bi], sic[bi])

    def i_wait(j, bi):
        off = pl.multiple_of(j * CH, CH)
        pltpu.make_async_copy(rows_hbm.at[pl.ds(rbase + off, CH)],
                              rb[bi], sir[bi]).wait()
        pltpu.make_async_copy(cols_hbm.at[pl.ds(cbase + off, CH)],
                              cb[bi], sic[bi]).wait()

    def g_start(bg, bi):
        pltpu.async_copy(mp_hbm.at[rb[bi]], gb[bg], sg[bg])

    def g_wait(bg, bi):
        pltpu.make_async_copy(mp_hbm.at[rb[bi]], gb[bg], sg[bg]).wait()

    def s_start(bg, bi):
        pltpu.async_copy(gb[bg], acc.at[cb[bi]], ss[bg], add=True)

    def s_wait(bg, bi):
        pltpu.make_async_copy(gb[bg], acc.at[cb[bi]], ss[bg]).wait()

    i_start(0, 0)
    i_start(1, 1)
    i_wait(0, 0)
    g_start(0, 0)
    i_start(2, 2)
    g_wait(0, 0)
    s_start(0, 0)
    i_wait(1, 1)
    g_start(1, 1)
    i_start(3, 3)
    g_wait(1, 1)
    s_start(1, 1)
    i_wait(2, 2)
    g_start(2, 2)

    def slot(j, bg, bi):
        g_wait(bg, bi)
        s_start(bg, bi)
        s_wait((bg + 1) % GB, (bi + 2) % IB)
        i_wait(j + 1, (bi + 1) % IB)
        g_start((bg + 1) % GB, (bi + 1) % IB)
        i_start(j + 2, (bi + 2) % IB)

    def step(t, carry):
        for k in range(12):
            j = 12 * t + 2 + k
            slot(j, (2 + k) % GB, (2 + k) % IB)
        return carry

    lax.fori_loop(0, (AGG_CHUNKS - 4) // 12, step, 0)
    j = AGG_CHUNKS - 2
    g_wait(j % GB, j % IB)
    s_start(j % GB, j % IB)
    s_wait((j - 2) % GB, (j - 2) % IB)
    i_wait(j + 1, (j + 1) % IB)
    g_start((j + 1) % GB, (j + 1) % IB)
    j = AGG_CHUNKS - 1
    g_wait(j % GB, j % IB)
    s_start(j % GB, j % IB)
    s_wait((j - 2) % GB, (j - 2) % IB)
    s_wait((j - 1) % GB, (j - 1) % IB)
    s_wait(j % GB, j % IB)

    plsc.subcore_barrier()
    pltpu.sync_copy(acc.at[pl.ds(s * ROWS_TILE, ROWS_TILE)],
                    out_hbm.at[c, pl.ds(s * ROWS_TILE, ROWS_TILE)])


@functools.partial(
    pl.kernel,
    out_type=jax.ShapeDtypeStruct((NC, N_PAD, H), jnp.float32),
    mesh=_mesh,
    scratch_types=(
        [pltpu.VMEM((CH,), jnp.int32)] * IB
        + [pltpu.VMEM((CH, H), jnp.float32)]
        + [pltpu.VMEM_SHARED((N_PAD, H), jnp.float32)]
        + [pltpu.SemaphoreType.DMA] * (2 * IB)
    ),
)
def _sc_deg(cols_hbm, ones_hbm, zeros_hbm, out_hbm, *scr):
    cb = scr[0:IB]
    vbuf = scr[IB]
    acc = scr[IB + 1]
    sems = scr[IB + 2:]
    sic = sems[0:IB]
    ss = sems[IB:]
    c = lax.axis_index("c")
    s = lax.axis_index("s")
    w = c * NS + s
    cbase = w * (DEG_CHUNKS * CH)
    pltpu.sync_copy(ones_hbm, vbuf)
    pltpu.sync_copy(zeros_hbm.at[pl.ds(s * ROWS_TILE, ROWS_TILE)],
                    acc.at[pl.ds(s * ROWS_TILE, ROWS_TILE)])
    plsc.subcore_barrier()

    def i_start(j, bi):
        off = pl.multiple_of(j * CH, CH)
        pltpu.async_copy(cols_hbm.at[pl.ds(cbase + off, CH)], cb[bi], sic[bi])

    def i_wait(j, bi):
        off = pl.multiple_of(j * CH, CH)
        pltpu.make_async_copy(cols_hbm.at[pl.ds(cbase + off, CH)],
                              cb[bi], sic[bi]).wait()

    def s_start(bi):
        pltpu.async_copy(vbuf, acc.at[cb[bi]], ss[bi], add=True)

    def s_wait(bi):
        pltpu.make_async_copy(vbuf, acc.at[cb[bi]], ss[bi]).wait()

    i_start(0, 0)
    i_start(1, 1)
    i_wait(0, 0)
    s_start(0)
    i_start(2, 2)
    i_wait(1, 1)
    s_start(1)
    i_start(3, 3)

    def step(t, carry):
        for k in range(IB):
            j = IB * t + 2 + k
            bi = (2 + k) % IB
            i_wait(j, bi)
            s_start(bi)
            s_wait((bi + 2) % IB)
            i_start(j + 2, (bi + 2) % IB)
        return carry

    lax.fori_loop(0, (DEG_CHUNKS - 4) // IB, step, 0)
    j = DEG_CHUNKS - 2
    i_wait(j, j % IB)
    s_start(j % IB)
    s_wait((j - 2) % IB)
    j = DEG_CHUNKS - 1
    i_wait(j, j % IB)
    s_start(j % IB)
    s_wait((j - 2) % IB)
    s_wait((j - 1) % IB)
    s_wait(j % IB)

    plsc.subcore_barrier()
    pltpu.sync_copy(acc.at[pl.ds(s * ROWS_TILE, ROWS_TILE)],
                    out_hbm.at[c, pl.ds(s * ROWS_TILE, ROWS_TILE)])


def _lin_body(x_ref, w_ref, b_ref, o_ref):
    o_ref[...] = x_ref[...] @ w_ref[...] + b_ref[...]


def _linear(x, w, b):
    k = x.shape[1]
    return pl.pallas_call(
        _lin_body,
        grid=(GRID,),
        in_specs=[
            pl.BlockSpec((BLK, k), lambda i: (i, 0)),
            pl.BlockSpec((k, D), lambda i: (0, 0)),
            pl.BlockSpec((1, D), lambda i: (0, 0)),
        ],
        out_specs=pl.BlockSpec((BLK, D), lambda i: (i, 0)),
        out_shape=jax.ShapeDtypeStruct((N, D), jnp.float32),
    )(x, w, b.reshape(1, D))


def _mp_body(h_ref, dinv_ref, w_ref, o_ref):
    m = (h_ref[...] @ w_ref[...]) * dinv_ref[...]
    o_ref[0] = m[:, :H]
    o_ref[1] = m[:, H:]


def _mp(h, dinv2, w):
    return pl.pallas_call(
        _mp_body,
        grid=(GRID,),
        in_specs=[
            pl.BlockSpec((BLK, D), lambda i: (i, 0)),
            pl.BlockSpec((BLK, 1), lambda i: (i, 0)),
            pl.BlockSpec((D, D), lambda i: (0, 0)),
        ],
        out_specs=pl.BlockSpec((NC, BLK, H), lambda i: (0, i, 0)),
        out_shape=jax.ShapeDtypeStruct((NC, N, H), jnp.float32),
    )(h, dinv2, w)


def _hnew_body(s_ref, r_ref, dinv_ref, o_ref):
    s_cat = jnp.concatenate([s_ref[0], s_ref[1]], axis=1)
    o_ref[...] = jnp.maximum(s_cat * dinv_ref[...] + r_ref[...], 0.0)


def _hnew(s, r, dinv2):
    return pl.pallas_call(
        _hnew_body,
        grid=(GRID,),
        in_specs=[
            pl.BlockSpec((NC, BLK, H), lambda i: (0, i, 0)),
            pl.BlockSpec((BLK, D), lambda i: (i, 0)),
            pl.BlockSpec((BLK, 1), lambda i: (i, 0)),
        ],
        out_specs=pl.BlockSpec((BLK, D), lambda i: (i, 0)),
        out_shape=jax.ShapeDtypeStruct((N, D), jnp.float32),
    )(s, r, dinv2)


def _heads_body(h_ref, w1_ref, b1_ref, w2_ref, b2_ref, w3_ref, b3_ref,
                mu_ref, std_ref):
    h = h_ref[...]
    ys = []
    for m in range(MIX):
        y1 = jnp.maximum(h @ w1_ref[m] + b1_ref[m], 0.0)
        y2 = jnp.maximum(y1 @ w2_ref[m] + b2_ref[m], 0.0)
        y3 = jnp.sum(y2 * w3_ref[m][None, :], axis=1, keepdims=True)
        ys.append(y3 + b3_ref[m, 0:1])
    mu = ys[0]
    for m in range(1, MIX):
        mu = mu + ys[m]
    mu = mu * (1.0 / MIX)
    var = (ys[0] - mu) ** 2
    for m in range(1, MIX):
        var = var + (ys[m] - mu) ** 2
    std = jnp.sqrt(var * (1.0 / MIX)) + 1e-5
    mu_ref[...] = mu
    std_ref[...] = std


def _heads(h, w1, b1, w2, b2, w3, b3):
    return pl.pallas_call(
        _heads_body,
        grid=(GRID,),
        in_specs=[
            pl.BlockSpec((BLK, D), lambda i: (i, 0)),
            pl.BlockSpec((MIX, D, D), lambda i: (0, 0, 0)),
            pl.BlockSpec((MIX, D), lambda i: (0, 0)),
            pl.BlockSpec((MIX, D, D), lambda i: (0, 0, 0)),
            pl.BlockSpec((MIX, D), lambda i: (0, 0)),
            pl.BlockSpec((MIX, D), lambda i: (0, 0)),
            pl.BlockSpec((MIX, 1), lambda i: (0, 0)),
        ],
        out_specs=[
            pl.BlockSpec((BLK, 1), lambda i: (i, 0)),
            pl.BlockSpec((BLK, 1), lambda i: (i, 0)),
        ],
        out_shape=[
            jax.ShapeDtypeStruct((N, 1), jnp.float32),
            jax.ShapeDtypeStruct((N, 1), jnp.float32),
        ],
    )(h, w1, b1, w2, b2, w3, b3)


def kernel(x, edge_index, input_idx, W_embed, b_embed, bn_gamma, bn_beta,
           bn_mean, bn_var, arma_Winit, arma_Wroot, arma_bias,
           head_W1, head_b1, head_W2, head_b2, head_W3, head_b3):
    f32 = jnp.float32
    row = edge_index[0]
    col = edge_index[1]

    scale = bn_gamma / jnp.sqrt(bn_var + 1e-5)
    w_emb = W_embed * scale[None, :]
    b_emb = (b_embed - bn_mean) * scale + bn_beta

    pad = E_PAD - E
    ar = jnp.arange(pad, dtype=jnp.int32)
    rows_p = jnp.concatenate([row, (ar * 97) % N])
    cols_p = jnp.concatenate([col, N + (ar % 16)])
    rows2 = jnp.stack([rows_p, rows_p + N]).reshape(NC * E_PAD)
    cols2 = cols_p
    zeros_h = jnp.zeros((N_PAD, H), f32)
    ones_h = jnp.ones((CH, H), f32)

    h = _linear(x, w_emb, b_emb)

    deg_parts = _sc_deg(cols2, ones_h, zeros_h)
    deg = deg_parts[0, :N, 0] + deg_parts[1, :N, 0]
    dinv2 = jnp.where(deg > 0, deg ** -0.5, 0.0).reshape(N, 1)

    for li in range(LAYER_N):
        mp = _mp(h, dinv2, arma_Winit[li])
        s = _sc_agg(rows2, cols2, mp.reshape(NC * N, H), zeros_h)
        r = _linear(h, arma_Wroot[li], arma_bias[li])
        h = _hnew(s[:, :N, :], r, dinv2)

    mu, std = _heads(h, head_W1, head_b1, head_W2, head_b2,
                     head_W3.reshape(MIX, D), head_b3.reshape(MIX, 1))
    return mu.reshape(N // 50, 50, 1), std.reshape(N // 50, 50, 1)


# speedup vs baseline: 12.7007x; 1.2960x over previous
import functools

import jax
import jax.numpy as jnp
from jax import lax
from jax.experimental import pallas as pl
from jax.experimental.pallas import tpu as pltpu
from jax.experimental.pallas import tpu_sc as plsc

N = 10000
E = 320000
D_IN = 128
D = 256
H = 128
MIX = 10
LAYER_N = 4

NC, NS = 2, 16
CH = 112
AGG_CHUNKS = 184
E_TILE = AGG_CHUNKS * CH
E_PAD = NS * E_TILE
DEG_CHUNKS = E_PAD // (NC * NS * CH)
N_PAD = 10112
ROWS_TILE = N_PAD // NS

BLK = 400
GRID = N // BLK

_mesh = plsc.VectorSubcoreMesh(core_axis_name="c", subcore_axis_name="s")


GB = 3
IB = 4


@functools.partial(
    pl.kernel,
    out_type=jax.ShapeDtypeStruct((NC, N_PAD, H), jnp.float32),
    mesh=_mesh,
    scratch_types=(
        [pltpu.VMEM((CH,), jnp.int32)] * IB
        + [pltpu.VMEM((CH,), jnp.int32)] * IB
        + [pltpu.VMEM((CH, H), jnp.float32)] * GB
        + [pltpu.VMEM_SHARED((N_PAD, H), jnp.float32)]
        + [pltpu.SemaphoreType.DMA] * (2 * IB + 2 * GB)
    ),
)
def _sc_agg(rows_hbm, cols_hbm, mp_hbm, zeros_hbm, out_hbm, *scr):
    rb = scr[0:IB]
    cb = scr[IB:2 * IB]
    gb = scr[2 * IB:2 * IB + GB]
    acc = scr[2 * IB + GB]
    sems = scr[2 * IB + GB + 1:]
    sir = sems[0:IB]
    sic = sems[IB:2 * IB]
    sg = sems[2 * IB:2 * IB + GB]
    ss = sems[2 * IB + GB:]
    c = lax.axis_index("c")
    s = lax.axis_index("s")
    w = c * NS + s
    rbase = w * E_TILE
    cbase = s * E_TILE
    pltpu.sync_copy(zeros_hbm.at[pl.ds(s * ROWS_TILE, ROWS_TILE)],
                    acc.at[pl.ds(s * ROWS_TILE, ROWS_TILE)])
    plsc.subcore_barrier()

    def i_start(j, bi):
        off = pl.multiple_of(j * CH, CH)
        pltpu.async_copy(rows_hbm.at[pl.ds(rbase + off, CH)], rb[bi], sir[bi])
        pltpu.async_copy(cols_hbm.at[pl.ds(cbase + off, CH)], cb[bi], sic[bi])

    def i_wait(j, bi):
        off = pl.multiple_of(j * CH, CH)
        pltpu.make_async_copy(rows_hbm.at[pl.ds(rbase + off, CH)],
                              rb[bi], sir[bi]).wait()
        pltpu.make_async_copy(cols_hbm.at[pl.ds(cbase + off, CH)],
                              cb[bi], sic[bi]).wait()

    def g_start(bg, bi):
        pltpu.async_copy(mp_hbm.at[rb[bi]], gb[bg], sg[bg])

    def g_wait(bg, bi):
        pltpu.make_async_copy(mp_hbm.at[rb[bi]], gb[bg], sg[bg]).wait()

    def s_start(bg, bi):
        pltpu.async_copy(gb[bg], acc.at[cb[bi]], ss[bg], add=True)

    def s_wait(bg, bi):
        pltpu.make_async_copy(gb[bg], acc.at[cb[bi]], ss[bg]).wait()

    i_start(0, 0)
    i_start(1, 1)
    i_start(2, 2)
    i_wait(0, 0)
    g_start(0, 0)
    i_wait(1, 1)
    g_start(1, 1)
    g_wait(0, 0)
    s_start(0, 0)
    i_wait(2, 2)
    g_start(2, 2)
    i_start(3, 3)
    g_wait(1, 1)
    s_start(1, 1)
    s_wait(0, 0)
    i_wait(3, 3)
    g_start(0, 3)
    i_start(4, 0)

    def slot(j, bg, bi):
        g_wait(bg, bi)
        s_start(bg, bi)
        s_wait((bg + 2) % GB, (bi + 3) % IB)
        i_wait(j + 2, (bi + 2) % IB)
        g_start((bg + 2) % GB, (bi + 2) % IB)
        i_start(j + 3, (bi + 3) % IB)

    STEADY = 12 * ((AGG_CHUNKS - 5) // 12)

    def step(t, carry):
        for k in range(12):
            j = 12 * t + 2 + k
            slot(j, (2 + k) % GB, (2 + k) % IB)
        return carry

    lax.fori_loop(0, STEADY // 12, step, 0)
    for j in range(STEADY + 2, AGG_CHUNKS):
        g_wait(j % GB, j % IB)
        s_start(j % GB, j % IB)
        s_wait((j - 1) % GB, (j - 1) % IB)
        if j + 2 < AGG_CHUNKS:
            i_wait(j + 2, (j + 2) % IB)
            g_start((j + 2) % GB, (j + 2) % IB)
        if j + 3 < AGG_CHUNKS:
            i_start(j + 3, (j + 3) % IB)
    s_wait((AGG_CHUNKS - 1) % GB, (AGG_CHUNKS - 1) % IB)

    plsc.subcore_barrier()
    pltpu.sync_copy(acc.at[pl.ds(s * ROWS_TILE, ROWS_TILE)],
                    out_hbm.at[c, pl.ds(s * ROWS_TILE, ROWS_TILE)])


@functools.partial(
    pl.kernel,
    out_type=jax.ShapeDtypeStruct((NC, N_PAD, H), jnp.float32),
    mesh=_mesh,
    scratch_types=(
        [pltpu.VMEM((CH,), jnp.int32)] * IB
        + [pltpu.VMEM((CH, H), jnp.float32)]
        + [pltpu.VMEM_SHARED((N_PAD, H), jnp.float32)]
        + [pltpu.SemaphoreType.DMA] * (2 * IB)
    ),
)
def _sc_deg(cols_hbm, ones_hbm, zeros_hbm, out_hbm, *scr):
    cb = scr[0:IB]
    vbuf = scr[IB]
    acc = scr[IB + 1]
    sems = scr[IB + 2:]
    sic = sems[0:IB]
    ss = sems[IB:]
    c = lax.axis_index("c")
    s = lax.axis_index("s")
    w = c * NS + s
    cbase = w * (DEG_CHUNKS * CH)
    pltpu.sync_copy(ones_hbm, vbuf)
    pltpu.sync_copy(zeros_hbm.at[pl.ds(s * ROWS_TILE, ROWS_TILE)],
                    acc.at[pl.ds(s * ROWS_TILE, ROWS_TILE)])
    plsc.subcore_barrier()

    def i_start(j, bi):
        off = pl.multiple_of(j * CH, CH)
        pltpu.async_copy(cols_hbm.at[pl.ds(cbase + off, CH)], cb[bi], sic[bi])

    def i_wait(j, bi):
        off = pl.multiple_of(j * CH, CH)
        pltpu.make_async_copy(cols_hbm.at[pl.ds(cbase + off, CH)],
                              cb[bi], sic[bi]).wait()

    def s_start(bi):
        pltpu.async_copy(vbuf, acc.at[cb[bi]], ss[bi], add=True)

    def s_wait(bi):
        pltpu.make_async_copy(vbuf, acc.at[cb[bi]], ss[bi]).wait()

    i_start(0, 0)
    i_start(1, 1)
    i_wait(0, 0)
    s_start(0)
    i_start(2, 2)
    i_wait(1, 1)
    s_start(1)
    i_start(3, 3)

    def step(t, carry):
        for k in range(IB):
            j = IB * t + 2 + k
            bi = (2 + k) % IB
            i_wait(j, bi)
            s_start(bi)
            s_wait((bi + 2) % IB)
            i_start(j + 2, (bi + 2) % IB)
        return carry

    lax.fori_loop(0, (DEG_CHUNKS - 4) // IB, step, 0)
    j = DEG_CHUNKS - 2
    i_wait(j, j % IB)
    s_start(j % IB)
    s_wait((j - 2) % IB)
    j = DEG_CHUNKS - 1
    i_wait(j, j % IB)
    s_start(j % IB)
    s_wait((j - 2) % IB)
    s_wait((j - 1) % IB)
    s_wait(j % IB)

    plsc.subcore_barrier()
    pltpu.sync_copy(acc.at[pl.ds(s * ROWS_TILE, ROWS_TILE)],
                    out_hbm.at[c, pl.ds(s * ROWS_TILE, ROWS_TILE)])


def _lin_body(x_ref, w_ref, b_ref, o_ref):
    o_ref[...] = x_ref[...] @ w_ref[...] + b_ref[...]


def _linear(x, w, b):
    k = x.shape[1]
    return pl.pallas_call(
        _lin_body,
        grid=(GRID,),
        in_specs=[
            pl.BlockSpec((BLK, k), lambda i: (i, 0)),
            pl.BlockSpec((k, D), lambda i: (0, 0)),
            pl.BlockSpec((1, D), lambda i: (0, 0)),
        ],
        out_specs=pl.BlockSpec((BLK, D), lambda i: (i, 0)),
        out_shape=jax.ShapeDtypeStruct((N, D), jnp.float32),
    )(x, w, b.reshape(1, D))


def _mp_body(h_ref, dinv_ref, w_ref, o_ref):
    m = (h_ref[...] @ w_ref[...]) * dinv_ref[...]
    o_ref[0] = m[:, :H]
    o_ref[1] = m[:, H:]


def _mp(h, dinv2, w):
    return pl.pallas_call(
        _mp_body,
        grid=(GRID,),
        in_specs=[
            pl.BlockSpec((BLK, D), lambda i: (i, 0)),
            pl.BlockSpec((BLK, 1), lambda i: (i, 0)),
            pl.BlockSpec((D, D), lambda i: (0, 0)),
        ],
        out_specs=pl.BlockSpec((NC, BLK, H), lambda i: (0, i, 0)),
        out_shape=jax.ShapeDtypeStruct((NC, N, H), jnp.float32),
    )(h, dinv2, w)


def _hnew_body(s_ref, r_ref, dinv_ref, o_ref):
    s_cat = jnp.concatenate([s_ref[0], s_ref[1]], axis=1)
    o_ref[...] = jnp.maximum(s_cat * dinv_ref[...] + r_ref[...], 0.0)


def _hnew(s, r, dinv2):
    return pl.pallas_call(
        _hnew_body,
        grid=(GRID,),
        in_specs=[
            pl.BlockSpec((NC, BLK, H), lambda i: (0, i, 0)),
            pl.BlockSpec((BLK, D), lambda i: (i, 0)),
            pl.BlockSpec((BLK, 1), lambda i: (i, 0)),
        ],
        out_specs=pl.BlockSpec((BLK, D), lambda i: (i, 0)),
        out_shape=jax.ShapeDtypeStruct((N, D), jnp.float32),
    )(s, r, dinv2)


def _heads_body(h_ref, w1_ref, b1_ref, w2_ref, b2_ref, w3_ref, b3_ref,
                mu_ref, std_ref):
    h = h_ref[...]
    ys = []
    for m in range(MIX):
        y1 = jnp.maximum(h @ w1_ref[m] + b1_ref[m], 0.0)
        y2 = jnp.maximum(y1 @ w2_ref[m] + b2_ref[m], 0.0)
        y3 = jnp.sum(y2 * w3_ref[m][None, :], axis=1, keepdims=True)
        ys.append(y3 + b3_ref[m, 0:1])
    mu = ys[0]
    for m in range(1, MIX):
        mu = mu + ys[m]
    mu = mu * (1.0 / MIX)
    var = (ys[0] - mu) ** 2
    for m in range(1, MIX):
        var = var + (ys[m] - mu) ** 2
    std = jnp.sqrt(var * (1.0 / MIX)) + 1e-5
    mu_ref[...] = mu
    std_ref[...] = std


def _heads(h, w1, b1, w2, b2, w3, b3):
    return pl.pallas_call(
        _heads_body,
        grid=(GRID,),
        in_specs=[
            pl.BlockSpec((BLK, D), lambda i: (i, 0)),
            pl.BlockSpec((MIX, D, D), lambda i: (0, 0, 0)),
            pl.BlockSpec((MIX, D), lambda i: (0, 0)),
            pl.BlockSpec((MIX, D, D), lambda i: (0, 0, 0)),
            pl.BlockSpec((MIX, D), lambda i: (0, 0)),
            pl.BlockSpec((MIX, D), lambda i: (0, 0)),
            pl.BlockSpec((MIX, 1), lambda i: (0, 0)),
        ],
        out_specs=[
            pl.BlockSpec((BLK, 1), lambda i: (i, 0)),
            pl.BlockSpec((BLK, 1), lambda i: (i, 0)),
        ],
        out_shape=[
            jax.ShapeDtypeStruct((N, 1), jnp.float32),
            jax.ShapeDtypeStruct((N, 1), jnp.float32),
        ],
    )(h, w1, b1, w2, b2, w3, b3)


def kernel(x, edge_index, input_idx, W_embed, b_embed, bn_gamma, bn_beta,
           bn_mean, bn_var, arma_Winit, arma_Wroot, arma_bias,
           head_W1, head_b1, head_W2, head_b2, head_W3, head_b3):
    f32 = jnp.float32
    row = edge_index[0]
    col = edge_index[1]

    scale = bn_gamma / jnp.sqrt(bn_var + 1e-5)
    w_emb = W_embed * scale[None, :]
    b_emb = (b_embed - bn_mean) * scale + bn_beta

    pad = E_PAD - E
    ar = jnp.arange(pad, dtype=jnp.int32)
    rows_p = jnp.concatenate([row, (ar * 97) % N])
    cols_p = jnp.concatenate([col, N + (ar % 16)])
    rows2 = jnp.stack([rows_p, rows_p + N]).reshape(NC * E_PAD)
    cols2 = cols_p
    zeros_h = jnp.zeros((N_PAD, H), f32)
    ones_h = jnp.ones((CH, H), f32)

    h = _linear(x, w_emb, b_emb)

    deg_parts = _sc_deg(cols2, ones_h, zeros_h)
    deg = deg_parts[0, :N, 0] + deg_parts[1, :N, 0]
    dinv2 = jnp.where(deg > 0, deg ** -0.5, 0.0).reshape(N, 1)

    for li in range(LAYER_N):
        mp = _mp(h, dinv2, arma_Winit[li])
        s = _sc_agg(rows2, cols2, mp.reshape(NC * N, H), zeros_h)
        r = _linear(h, arma_Wroot[li], arma_bias[li])
        h = _hnew(s[:, :N, :], r, dinv2)

    mu, std = _heads(h, head_W1, head_b1, head_W2, head_b2,
                     head_W3.reshape(MIX, D), head_b3.reshape(MIX, 1))
    return mu.reshape(N // 50, 50, 1), std.reshape(N // 50, 50, 1)


# speedup vs baseline: 13.8512x; 1.0906x over previous
import functools

import jax
import jax.numpy as jnp
from jax import lax
from jax.experimental import pallas as pl
from jax.experimental.pallas import tpu as pltpu
from jax.experimental.pallas import tpu_sc as plsc

N = 10000
E = 320000
D_IN = 128
D = 256
H = 128
MIX = 10
LAYER_N = 4

NC, NS = 2, 16
CH = 112
AGG_CHUNKS = 184
E_TILE = AGG_CHUNKS * CH
E_PAD = NS * E_TILE
DEG_CHUNKS = E_PAD // (NC * NS * CH)
N_PAD = 10112
ROWS_TILE = N_PAD // NS

BLK = 400
GRID = N // BLK

_mesh = plsc.VectorSubcoreMesh(core_axis_name="c", subcore_axis_name="s")


GB = 3
IB = 4


@functools.partial(
    pl.kernel,
    out_type=jax.ShapeDtypeStruct((NC, N_PAD, H), jnp.float32),
    mesh=_mesh,
    scratch_types=(
        [pltpu.VMEM((CH,), jnp.int32)] * IB
        + [pltpu.VMEM((CH,), jnp.int32)] * IB
        + [pltpu.VMEM((CH, H), jnp.float32)] * GB
        + [pltpu.VMEM_SHARED((N_PAD, H), jnp.float32)]
        + [pltpu.SemaphoreType.DMA] * (2 * IB + 2 * GB)
    ),
)
def _sc_agg(rows_hbm, cols_hbm, mp_hbm, zeros_hbm, out_hbm, *scr):
    rb = scr[0:IB]
    cb = scr[IB:2 * IB]
    gb = scr[2 * IB:2 * IB + GB]
    acc = scr[2 * IB + GB]
    sems = scr[2 * IB + GB + 1:]
    sir = sems[0:IB]
    sic = sems[IB:2 * IB]
    sg = sems[2 * IB:2 * IB + GB]
    ss = sems[2 * IB + GB:]
    c = lax.axis_index("c")
    s = lax.axis_index("s")
    w = c * NS + s
    rbase = w * E_TILE
    cbase = s * E_TILE
    pltpu.sync_copy(zeros_hbm.at[pl.ds(s * ROWS_TILE, ROWS_TILE)],
                    acc.at[pl.ds(s * ROWS_TILE, ROWS_TILE)])
    plsc.subcore_barrier()

    def i_start(j, bi):
        off = pl.multiple_of(j * CH, CH)
        pltpu.async_copy(rows_hbm.at[pl.ds(rbase + off, CH)], rb[bi], sir[bi])
        pltpu.async_copy(cols_hbm.at[pl.ds(cbase + off, CH)], cb[bi], sic[bi])

    def i_wait(j, bi):
        off = pl.multiple_of(j * CH, CH)
        pltpu.make_async_copy(rows_hbm.at[pl.ds(rbase + off, CH)],
                              rb[bi], sir[bi]).wait()
        pltpu.make_async_copy(cols_hbm.at[pl.ds(cbase + off, CH)],
                              cb[bi], sic[bi]).wait()

    def g_start(bg, bi):
        pltpu.async_copy(mp_hbm.at[rb[bi]], gb[bg], sg[bg])

    def g_wait(bg, bi):
        pltpu.make_async_copy(mp_hbm.at[rb[bi]], gb[bg], sg[bg]).wait()

    def s_start(bg, bi):
        pltpu.async_copy(gb[bg], acc.at[cb[bi]], ss[bg], add=True)

    def s_wait(bg, bi):
        pltpu.make_async_copy(gb[bg], acc.at[cb[bi]], ss[bg]).wait()

    i_start(0, 0)
    i_start(1, 1)
    i_start(2, 2)
    i_wait(0, 0)
    g_start(0, 0)
    i_wait(1, 1)
    g_start(1, 1)
    g_wait(0, 0)
    s_start(0, 0)
    i_wait(2, 2)
    g_start(2, 2)
    i_start(3, 3)
    g_wait(1, 1)
    s_start(1, 1)
    s_wait(0, 0)
    i_wait(3, 3)
    g_start(0, 3)
    i_start(4, 0)

    def slot(j, bg, bi):
        g_wait(bg, bi)
        s_start(bg, bi)
        s_wait((bg + 2) % GB, (bi + 3) % IB)
        i_wait(j + 2, (bi + 2) % IB)
        g_start((bg + 2) % GB, (bi + 2) % IB)
        i_start(j + 3, (bi + 3) % IB)

    STEADY = 12 * ((AGG_CHUNKS - 5) // 12)

    def step(t, carry):
        for k in range(12):
            j = 12 * t + 2 + k
            slot(j, (2 + k) % GB, (2 + k) % IB)
        return carry

    lax.fori_loop(0, STEADY // 12, step, 0)
    for j in range(STEADY + 2, AGG_CHUNKS):
        g_wait(j % GB, j % IB)
        s_start(j % GB, j % IB)
        s_wait((j - 1) % GB, (j - 1) % IB)
        if j + 2 < AGG_CHUNKS:
            i_wait(j + 2, (j + 2) % IB)
            g_start((j + 2) % GB, (j + 2) % IB)
        if j + 3 < AGG_CHUNKS:
            i_start(j + 3, (j + 3) % IB)
    s_wait((AGG_CHUNKS - 1) % GB, (AGG_CHUNKS - 1) % IB)

    plsc.subcore_barrier()
    pltpu.sync_copy(acc.at[pl.ds(s * ROWS_TILE, ROWS_TILE)],
                    out_hbm.at[c, pl.ds(s * ROWS_TILE, ROWS_TILE)])


@functools.partial(
    pl.kernel,
    out_type=jax.ShapeDtypeStruct((NC, N_PAD, H), jnp.float32),
    mesh=_mesh,
    scratch_types=(
        [pltpu.VMEM((CH,), jnp.int32)] * IB
        + [pltpu.VMEM((CH, H), jnp.float32)]
        + [pltpu.VMEM_SHARED((N_PAD, H), jnp.float32)]
        + [pltpu.SemaphoreType.DMA] * (2 * IB)
    ),
)
def _sc_deg(cols_hbm, ones_hbm, zeros_hbm, out_hbm, *scr):
    cb = scr[0:IB]
    vbuf = scr[IB]
    acc = scr[IB + 1]
    sems = scr[IB + 2:]
    sic = sems[0:IB]
    ss = sems[IB:]
    c = lax.axis_index("c")
    s = lax.axis_index("s")
    w = c * NS + s
    cbase = w * (DEG_CHUNKS * CH)
    pltpu.sync_copy(ones_hbm, vbuf)
    pltpu.sync_copy(zeros_hbm.at[pl.ds(s * ROWS_TILE, ROWS_TILE)],
                    acc.at[pl.ds(s * ROWS_TILE, ROWS_TILE)])
    plsc.subcore_barrier()

    def i_start(j, bi):
        off = pl.multiple_of(j * CH, CH)
        pltpu.async_copy(cols_hbm.at[pl.ds(cbase + off, CH)], cb[bi], sic[bi])

    def i_wait(j, bi):
        off = pl.multiple_of(j * CH, CH)
        pltpu.make_async_copy(cols_hbm.at[pl.ds(cbase + off, CH)],
                              cb[bi], sic[bi]).wait()

    def s_start(bi):
        pltpu.async_copy(vbuf, acc.at[cb[bi]], ss[bi], add=True)

    def s_wait(bi):
        pltpu.make_async_copy(vbuf, acc.at[cb[bi]], ss[bi]).wait()

    i_start(0, 0)
    i_start(1, 1)
    i_wait(0, 0)
    s_start(0)
    i_start(2, 2)
    i_wait(1, 1)
    s_start(1)
    i_start(3, 3)

    def step(t, carry):
        for k in range(IB):
            j = IB * t + 2 + k
            bi = (2 + k) % IB
            i_wait(j, bi)
            s_start(bi)
            s_wait((bi + 2) % IB)
            i_start(j + 2, (bi + 2) % IB)
        return carry

    lax.fori_loop(0, (DEG_CHUNKS - 4) // IB, step, 0)
    j = DEG_CHUNKS - 2
    i_wait(j, j % IB)
    s_start(j % IB)
    s_wait((j - 2) % IB)
    j = DEG_CHUNKS - 1
    i_wait(j, j % IB)
    s_start(j % IB)
    s_wait((j - 2) % IB)
    s_wait((j - 1) % IB)
    s_wait(j % IB)

    plsc.subcore_barrier()
    pltpu.sync_copy(acc.at[pl.ds(s * ROWS_TILE, ROWS_TILE)],
                    out_hbm.at[c, pl.ds(s * ROWS_TILE, ROWS_TILE)])


def _lin_body(x_ref, w_ref, b_ref, o_ref):
    o_ref[...] = x_ref[...] @ w_ref[...] + b_ref[...]


def _linear(x, w, b):
    k = x.shape[1]
    return pl.pallas_call(
        _lin_body,
        grid=(GRID,),
        in_specs=[
            pl.BlockSpec((BLK, k), lambda i: (i, 0)),
            pl.BlockSpec((k, D), lambda i: (0, 0)),
            pl.BlockSpec((1, D), lambda i: (0, 0)),
        ],
        out_specs=pl.BlockSpec((BLK, D), lambda i: (i, 0)),
        out_shape=jax.ShapeDtypeStruct((N, D), jnp.float32),
    )(x, w, b.reshape(1, D))


def _layer0_body(h_ref, dinv_ref, wcat_ref, b_ref, mp_ref, r_ref):
    mm = h_ref[...] @ wcat_ref[...]
    m = mm[:, :D] * dinv_ref[...]
    mp_ref[0] = m[:, :H]
    mp_ref[1] = m[:, H:]
    r_ref[...] = mm[:, D:] + b_ref[...]


def _layer_body(s_ref, rp_ref, dinv_ref, wcat_ref, b_ref, mp_ref, r_ref):
    s_cat = jnp.concatenate([s_ref[0], s_ref[1]], axis=1)
    hb = jnp.maximum(s_cat * dinv_ref[...] + rp_ref[...], 0.0)
    mm = hb @ wcat_ref[...]
    m = mm[:, :D] * dinv_ref[...]
    mp_ref[0] = m[:, :H]
    mp_ref[1] = m[:, H:]
    r_ref[...] = mm[:, D:] + b_ref[...]


_LAYER_OUT_SPECS = [
    pl.BlockSpec((NC, BLK, H), lambda i: (0, i, 0)),
    pl.BlockSpec((BLK, D), lambda i: (i, 0)),
]
_LAYER_OUT_SHAPE = [
    jax.ShapeDtypeStruct((NC, N, H), jnp.float32),
    jax.ShapeDtypeStruct((N, D), jnp.float32),
]


def _layer0(h, dinv2, wcat, b):
    return pl.pallas_call(
        _layer0_body,
        grid=(GRID,),
        in_specs=[
            pl.BlockSpec((BLK, D), lambda i: (i, 0)),
            pl.BlockSpec((BLK, 1), lambda i: (i, 0)),
            pl.BlockSpec((D, 2 * D), lambda i: (0, 0)),
            pl.BlockSpec((1, D), lambda i: (0, 0)),
        ],
        out_specs=_LAYER_OUT_SPECS,
        out_shape=_LAYER_OUT_SHAPE,
    )(h, dinv2, wcat, b.reshape(1, D))


def _layer(s, rp, dinv2, wcat, b):
    return pl.pallas_call(
        _layer_body,
        grid=(GRID,),
        in_specs=[
            pl.BlockSpec((NC, BLK, H), lambda i: (0, i, 0)),
            pl.BlockSpec((BLK, D), lambda i: (i, 0)),
            pl.BlockSpec((BLK, 1), lambda i: (i, 0)),
            pl.BlockSpec((D, 2 * D), lambda i: (0, 0)),
            pl.BlockSpec((1, D), lambda i: (0, 0)),
        ],
        out_specs=_LAYER_OUT_SPECS,
        out_shape=_LAYER_OUT_SHAPE,
    )(s, rp, dinv2, wcat, b.reshape(1, D))


def _heads_body(s_ref, rp_ref, dinv_ref, w1_ref, b1_ref, w2_ref, b2_ref,
                w3_ref, b3_ref, mu_ref, std_ref):
    s_cat = jnp.concatenate([s_ref[0], s_ref[1]], axis=1)
    h = jnp.maximum(s_cat * dinv_ref[...] + rp_ref[...], 0.0)
    ys = []
    for m in range(MIX):
        y1 = jnp.maximum(h @ w1_ref[m] + b1_ref[m], 0.0)
        y2 = jnp.maximum(y1 @ w2_ref[m] + b2_ref[m], 0.0)
        y3 = jnp.sum(y2 * w3_ref[m][None, :], axis=1, keepdims=True)
        ys.append(y3 + b3_ref[m, 0:1])
    mu = ys[0]
    for m in range(1, MIX):
        mu = mu + ys[m]
    mu = mu * (1.0 / MIX)
    var = (ys[0] - mu) ** 2
    for m in range(1, MIX):
        var = var + (ys[m] - mu) ** 2
    std = jnp.sqrt(var * (1.0 / MIX)) + 1e-5
    mu_ref[...] = mu
    std_ref[...] = std


def _heads(s, rp, dinv2, w1, b1, w2, b2, w3, b3):
    return pl.pallas_call(
        _heads_body,
        grid=(GRID,),
        in_specs=[
            pl.BlockSpec((NC, BLK, H), lambda i: (0, i, 0)),
            pl.BlockSpec((BLK, D), lambda i: (i, 0)),
            pl.BlockSpec((BLK, 1), lambda i: (i, 0)),
            pl.BlockSpec((MIX, D, D), lambda i: (0, 0, 0)),
            pl.BlockSpec((MIX, D), lambda i: (0, 0)),
            pl.BlockSpec((MIX, D, D), lambda i: (0, 0, 0)),
            pl.BlockSpec((MIX, D), lambda i: (0, 0)),
            pl.BlockSpec((MIX, D), lambda i: (0, 0)),
            pl.BlockSpec((MIX, 1), lambda i: (0, 0)),
        ],
        out_specs=[
            pl.BlockSpec((BLK, 1), lambda i: (i, 0)),
            pl.BlockSpec((BLK, 1), lambda i: (i, 0)),
        ],
        out_shape=[
            jax.ShapeDtypeStruct((N, 1), jnp.float32),
            jax.ShapeDtypeStruct((N, 1), jnp.float32),
        ],
    )(s, rp, dinv2, w1, b1, w2, b2, w3, b3)


def kernel(x, edge_index, input_idx, W_embed, b_embed, bn_gamma, bn_beta,
           bn_mean, bn_var, arma_Winit, arma_Wroot, arma_bias,
           head_W1, head_b1, head_W2, head_b2, head_W3, head_b3):
    f32 = jnp.float32
    row = edge_index[0]
    col = edge_index[1]

    scale = bn_gamma / jnp.sqrt(bn_var + 1e-5)
    w_emb = W_embed * scale[None, :]
    b_emb = (b_embed - bn_mean) * scale + bn_beta

    pad = E_PAD - E
    ar = jnp.arange(pad, dtype=jnp.int32)
    rows_p = jnp.concatenate([row, (ar * 97) % N])
    cols_p = jnp.concatenate([col, N + (ar % 16)])
    rows2 = jnp.stack([rows_p, rows_p + N]).reshape(NC * E_PAD)
    cols2 = cols_p
    zeros_h = jnp.zeros((N_PAD, H), f32)
    ones_h = jnp.ones((CH, H), f32)

    h = _linear(x, w_emb, b_emb)

    deg_parts = _sc_deg(cols2, ones_h, zeros_h)
    deg = deg_parts[0, :N, 0] + deg_parts[1, :N, 0]
    dinv2 = jnp.where(deg > 0, deg ** -0.5, 0.0).reshape(N, 1)

    wcat = jnp.concatenate([arma_Winit, arma_Wroot], axis=2)

    mp, r = _layer0(h, dinv2, wcat[0], arma_bias[0])
    s = _sc_agg(rows2, cols2, mp.reshape(NC * N, H), zeros_h)
    for li in range(1, LAYER_N):
        mp, r = _layer(s, r, dinv2, wcat[li], arma_bias[li])
        s = _sc_agg(rows2, cols2, mp.reshape(NC * N, H), zeros_h)

    mu, std = _heads(s, r, dinv2, head_W1, head_b1, head_W2, head_b2,
                     head_W3.reshape(MIX, D), head_b3.reshape(MIX, 1))
    return mu.reshape(N // 50, 50, 1), std.reshape(N // 50, 50, 1)
